# baseline (speedup 1.0000x reference)
import functools

import jax
import jax.numpy as jnp
from jax.experimental import pallas as pl
from jax.experimental.pallas import tpu as pltpu

N_S = 10000
N_T = 10000
D_FEAT = 64
NUM_CLASSES = 22
TEMP = 2.0

BS = 200
NT_PAD = 10240
TT = 2048
N_SB = N_S // BS


def _log_softmax(x):
    m = jnp.max(x, axis=1, keepdims=True)
    y = x - m
    return y - jnp.log(jnp.sum(jnp.exp(y), axis=1, keepdims=True))


def _body(s_aug_ref, t_aug_t_ref, s_feat_ref, t_feat_ref, w_ref, b_ref,
          wt_ref, bt_ref, seg_ref, ce_ref, kl_ref, t_logits_ref):
    pid = pl.program_id(0)

    @pl.when(pid == 0)
    def _compute_teacher_logits():
        t_logits_ref[...] = (
            jnp.dot(t_feat_ref[...], wt_ref[...],
                    preferred_element_type=jnp.float32) + bt_ref[...])

    s_aug = s_aug_ref[...]

    def t_tile(i, carry):
        best, bidx = carry
        t_blk = t_aug_t_ref[:, pl.ds(i * TT, TT)]
        sc = jnp.dot(s_aug, t_blk, preferred_element_type=jnp.float32)
        m = jnp.max(sc, axis=1, keepdims=True)
        col = jax.lax.broadcasted_iota(jnp.int32, (BS, TT), 1) + i * TT
        li = jnp.min(jnp.where(sc == m, col, jnp.int32(2**30)),
                     axis=1, keepdims=True)
        upd = m > best
        return jnp.where(upd, m, best), jnp.where(upd, li, bidx)

    best0 = jnp.full((BS, 1), -jnp.inf, dtype=jnp.float32)
    bidx0 = jnp.zeros((BS, 1), dtype=jnp.int32)
    _, bidx = jax.lax.fori_loop(0, NT_PAD // TT, t_tile, (best0, bidx0))

    col_full = jax.lax.broadcasted_iota(jnp.int32, (BS, N_T), 1)
    onehot = (col_full == bidx).astype(jnp.float32)
    matched = jnp.dot(onehot, t_logits_ref[...],
                      preferred_element_type=jnp.float32)

    sl = (jnp.dot(s_feat_ref[...], w_ref[...],
                  preferred_element_type=jnp.float32) + b_ref[...])
    logp = _log_softmax(sl)
    seg = seg_ref[0, 0, :]
    cls = jax.lax.broadcasted_iota(jnp.int32, (BS, NUM_CLASSES), 1)
    seg_oh = cls == seg[:, None]
    ce_sum = -jnp.sum(jnp.where(seg_oh, logp, 0.0))

    slp = _log_softmax(sl / TEMP)
    tlp = _log_softmax(matched / TEMP)
    tp = jnp.exp(tlp)
    kl_sum = jnp.sum(tp * (tlp - slp))

    ce_ref[...] = jnp.broadcast_to(ce_sum, (1, 1, 128))
    kl_ref[...] = jnp.broadcast_to(kl_sum, (1, 1, 128))


@jax.jit
def kernel(s_feat, t_feat, student_coords, teacher_coords, W, b, Wt, bt,
           segment):
    s_aug = jnp.concatenate(
        [student_coords, jnp.ones((N_S, 1), jnp.float32)], axis=1)
    t2 = jnp.sum(teacher_coords * teacher_coords, axis=1)
    t_aug_t = jnp.concatenate([teacher_coords.T, (-0.5 * t2)[None, :]], axis=0)
    pad = jnp.zeros((4, NT_PAD - N_T), jnp.float32).at[3, :].set(-1e30)
    t_aug_t = jnp.concatenate([t_aug_t, pad], axis=1)

    seg3 = segment.astype(jnp.int32).reshape(N_SB, 1, BS)
    b2 = b.reshape(1, NUM_CLASSES)
    bt2 = bt.reshape(1, NUM_CLASSES)

    ce_part, kl_part = pl.pallas_call(
        _body,
        grid=(N_SB,),
        in_specs=[
            pl.BlockSpec((BS, 4), lambda i: (i, 0)),
            pl.BlockSpec((4, NT_PAD), lambda i: (0, 0)),
            pl.BlockSpec((BS, D_FEAT), lambda i: (i, 0)),
            pl.BlockSpec((N_T, D_FEAT), lambda i: (0, 0)),
            pl.BlockSpec((D_FEAT, NUM_CLASSES), lambda i: (0, 0)),
            pl.BlockSpec((1, NUM_CLASSES), lambda i: (0, 0)),
            pl.BlockSpec((D_FEAT, NUM_CLASSES), lambda i: (0, 0)),
            pl.BlockSpec((1, NUM_CLASSES), lambda i: (0, 0)),
            pl.BlockSpec((1, 1, BS), lambda i: (i, 0, 0)),
        ],
        out_specs=[
            pl.BlockSpec((1, 1, 128), lambda i: (i, 0, 0)),
            pl.BlockSpec((1, 1, 128), lambda i: (i, 0, 0)),
        ],
        out_shape=[
            jax.ShapeDtypeStruct((N_SB, 1, 128), jnp.float32),
            jax.ShapeDtypeStruct((N_SB, 1, 128), jnp.float32),
        ],
        scratch_shapes=[pltpu.VMEM((N_T, NUM_CLASSES), jnp.float32)],
    )(s_aug, t_aug_t, s_feat, t_feat, W, b2, Wt, bt2, seg3)

    seg_loss = jnp.sum(ce_part[:, 0, 0]) / N_S
    kl1 = jnp.sum(kl_part[:, 0, 0]) / N_S * (TEMP ** 2)
    kl_loss = 0.2 * kl1
    total_loss = seg_loss + kl_loss
    return (total_loss, seg_loss, kl_loss)
